# baseline (speedup 1.0000x reference)
import functools

import jax
import jax.numpy as jnp
from jax.experimental import pallas as pl
from jax.experimental.pallas import tpu as pltpu

UNITS = 64
MAX_N = 3
SHF = 16
RBF = MAX_N * SHF
CUTOFF = 5.0

ROWS = 4096


def _sig(x):
    return jax.nn.sigmoid(x)


def _silu(x):
    return x * _sig(x)


def _dot(a, b):
    return jax.lax.dot_general(a, b, (((1,), (0,)), ((), ())),
                               preferred_element_type=jnp.float32)


def _basis_body(v1_ref, v2_ref, tb_ref):
    v1 = v1_ref[...]
    v2 = v2_ref[...]
    rows = v1.shape[0]
    d2 = jnp.sum(v2 * v2, axis=1, keepdims=True)
    tlen = jnp.sqrt(d2)
    n1 = v1 / (jnp.sqrt(jnp.sum(v1 * v1, axis=1, keepdims=True)) + 1e-8)
    n2 = v2 / (tlen + 1e-8)
    x = jnp.clip(jnp.sum(n1 * n2, axis=1, keepdims=True), -1.0, 1.0)
    theta = jax.lax.atan2(jnp.sqrt(jnp.maximum(1.0 - x * x, 0.0)), x)
    k = jax.lax.broadcasted_iota(jnp.int32, (rows, RBF), 1)
    lidx = jnp.mod(k, SHF).astype(jnp.float32)
    nidx = (k // SHF + 1).astype(jnp.float32)
    r = tlen
    coef = jnp.sqrt(2.0 / CUTOFF)
    s = jnp.pi / CUTOFF
    inv = coef / (r + 1e-8)
    tb_ref[...] = (jnp.sin(nidx * (s * r)) * inv) * jnp.cos(lidx * theta)


def _basis(v1, v2):
    t = v1.shape[0]
    grid = pl.cdiv(t, ROWS)
    return pl.pallas_call(
        _basis_body,
        grid=(grid,),
        in_specs=[
            pl.BlockSpec((ROWS, 3), lambda i: (i, 0)),
            pl.BlockSpec((ROWS, 3), lambda i: (i, 0)),
        ],
        out_specs=pl.BlockSpec((ROWS, RBF), lambda i: (i, 0)),
        out_shape=jax.ShapeDtypeStruct((t, RBF), jnp.float32),
    )(v1, v2)


def _bond_body(agg_ref, bond_ref, aS_ref, aR_ref,
               wl_ref, bl_ref, wg_ref, bg_ref,
               wb1_ref, wb2_ref, wb3_ref, bb_ref,
               wa_ref, ba_ref, wag_ref, bag_ref,
               bond_out, m_out):
    agg = agg_ref[...]
    bond = bond_ref[...]
    g1 = _silu(_dot(agg, wl_ref[...]) + bl_ref[...])
    g2 = _sig(_dot(agg, wg_ref[...]) + bg_ref[...])
    bond1 = bond + g1 * g2
    pre = (_dot(aS_ref[...], wb1_ref[...]) + _dot(aR_ref[...], wb2_ref[...])
           + _dot(bond1, wb3_ref[...]) + bb_ref[...])
    bond2 = bond1 + _silu(pre)
    m = _silu(_dot(bond2, wa_ref[...]) + ba_ref[...]) * _sig(
        _dot(bond2, wag_ref[...]) + bag_ref[...])
    bond_out[...] = bond2
    m_out[...] = m


def _bond_block(agg, bond, aS, aR, blk):
    e = bond.shape[0]
    grid = pl.cdiv(e, ROWS)
    wb = blk['Wb']
    row = pl.BlockSpec((ROWS, None), lambda i: (i, 0))
    full = lambda r, c: pl.BlockSpec((r, c), lambda i: (0, 0))
    return pl.pallas_call(
        _bond_body,
        grid=(grid,),
        in_specs=[
            pl.BlockSpec((ROWS, RBF), lambda i: (i, 0)),
            pl.BlockSpec((ROWS, UNITS), lambda i: (i, 0)),
            pl.BlockSpec((ROWS, UNITS), lambda i: (i, 0)),
            pl.BlockSpec((ROWS, UNITS), lambda i: (i, 0)),
            full(RBF, UNITS), full(1, UNITS),
            full(RBF, UNITS), full(1, UNITS),
            full(UNITS, UNITS), full(UNITS, UNITS), full(UNITS, UNITS),
            full(1, UNITS),
            full(UNITS, UNITS), full(1, UNITS),
            full(UNITS, UNITS), full(1, UNITS),
        ],
        out_specs=[
            pl.BlockSpec((ROWS, UNITS), lambda i: (i, 0)),
            pl.BlockSpec((ROWS, UNITS), lambda i: (i, 0)),
        ],
        out_shape=[
            jax.ShapeDtypeStruct((e, UNITS), jnp.float32),
            jax.ShapeDtypeStruct((e, UNITS), jnp.float32),
        ],
    )(agg, bond, aS, aR,
      blk['Wl'], blk['bl'][None, :], blk['Wg'], blk['bg'][None, :],
      wb[:UNITS], wb[UNITS:2 * UNITS], wb[2 * UNITS:], blk['bb'][None, :],
      blk['Wa'], blk['ba'][None, :], blk['Wag'], blk['bag'][None, :])


def kernel(atom_positions, atom_types, bond_atom_indices,
           triple_bond_indices, batch_ids, params):
    pos = atom_positions
    n = pos.shape[0]
    e = bond_atom_indices.shape[0]
    t = triple_bond_indices.shape[0]
    perm_e = jnp.argsort(bond_atom_indices[:, 1])
    sender = bond_atom_indices[perm_e, 0]
    receiver = bond_atom_indices[perm_e, 1]
    inv_e = jnp.zeros((e,), jnp.int32).at[perm_e].set(
        jnp.arange(e, dtype=jnp.int32), unique_indices=True,
        indices_are_sorted=False, mode='promise_in_bounds')
    b1 = inv_e[triple_bond_indices[:, 0]]
    perm_t = jnp.argsort(b1)
    b1s = b1[perm_t]
    b2s = inv_e[triple_bond_indices[perm_t, 1]]
    vec = pos[receiver] - pos[sender]
    blen = jnp.sqrt(jnp.sum(vec * vec, axis=1, keepdims=True))
    v1 = vec[b1s]
    v2 = vec[b2s]
    tb = _basis(v1, v2)
    apex = receiver[b2s]
    atom = params['emb'][atom_types]
    bond = _silu(blen @ params['Wbp'] + params['bbp'])
    for blk in params['blocks']:
        upd = _sig(atom @ blk['Wu'] + blk['bu'])
        msg = tb * upd[apex]
        agg = jax.ops.segment_sum(msg, b1s, num_segments=e,
                                  indices_are_sorted=True)
        bond, m = _bond_block(agg, bond, atom[sender], atom[receiver], blk)
        atom = atom + jax.ops.segment_sum(m, receiver, num_segments=n,
                                          indices_are_sorted=True)
    per_atom = atom @ params['Wr'] + params['br']
    energy = jax.ops.segment_sum(per_atom, batch_ids, num_segments=128,
                                 indices_are_sorted=True)
    return energy


# speedup vs baseline: 1.4569x; 1.4569x over previous
import functools

import jax
import jax.numpy as jnp
from jax.experimental import pallas as pl
from jax.experimental.pallas import tpu as pltpu
from jax.experimental.pallas import tpu_sc as plsc

UNITS = 64
MAX_N = 3
SHF = 16
RBF = MAX_N * SHF
CUTOFF = 5.0

ROWS = 4096


@functools.lru_cache(maxsize=None)
def _make_sc_gather(D, B, CH):
    info = plsc.get_sparse_core_info()
    nc, ns = info.num_cores, info.num_subcores
    nw = nc * ns
    assert B % nw == 0
    bpw = B // nw
    assert bpw % CH == 0 and CH % 8 == 0
    nch = bpw // CH
    mesh = plsc.VectorSubcoreMesh(core_axis_name="c", subcore_axis_name="s")

    def gk(table, idx, out, idx_v, rows_v, sem):
        wid = jax.lax.axis_index("s") * nc + jax.lax.axis_index("c")
        base = wid * bpw

        def body(c, carry):
            off = base + c * CH
            pltpu.sync_copy(idx.at[pl.ds(off, CH)], idx_v)
            pltpu.async_copy(table.at[idx_v], rows_v, sem).wait()
            pltpu.sync_copy(rows_v, out.at[pl.ds(off, CH)])
            return carry

        jax.lax.fori_loop(0, nch, body, 0)

    def call(table, idx):
        k = functools.partial(
            pl.kernel, mesh=mesh,
            out_type=jax.ShapeDtypeStruct((B, D), jnp.float32),
            scratch_types=[
                pltpu.VMEM((CH,), jnp.int32),
                pltpu.VMEM((CH, D), jnp.float32),
                pltpu.SemaphoreType.DMA,
            ],
        )(gk)
        return k(table, idx)

    return call


def _sc_gather(table, idx, CH=1000):
    assert table.shape[1] == 128
    return _make_sc_gather(128, idx.shape[0], CH)(table, idx)


def _sig(x):
    return jax.nn.sigmoid(x)


def _silu(x):
    return x * _sig(x)


def _dot(a, b):
    return jax.lax.dot_general(a, b, (((1,), (0,)), ((), ())),
                               preferred_element_type=jnp.float32)


def _basis_body(v1_ref, v2_ref, tb_ref):
    v1 = v1_ref[...]
    v2 = v2_ref[...]
    rows = v1.shape[0]
    d2 = jnp.sum(v2 * v2, axis=1, keepdims=True)
    tlen = jnp.sqrt(d2)
    n1 = v1 / (jnp.sqrt(jnp.sum(v1 * v1, axis=1, keepdims=True)) + 1e-8)
    n2 = v2 / (tlen + 1e-8)
    x = jnp.clip(jnp.sum(n1 * n2, axis=1, keepdims=True), -1.0, 1.0)
    theta = jax.lax.atan2(jnp.sqrt(jnp.maximum(1.0 - x * x, 0.0)), x)
    k = jax.lax.broadcasted_iota(jnp.int32, (rows, RBF), 1)
    lidx = jnp.mod(k, SHF).astype(jnp.float32)
    nidx = (k // SHF + 1).astype(jnp.float32)
    r = tlen
    coef = jnp.sqrt(2.0 / CUTOFF)
    s = jnp.pi / CUTOFF
    inv = coef / (r + 1e-8)
    tb_ref[...] = (jnp.sin(nidx * (s * r)) * inv) * jnp.cos(lidx * theta)


def _basis(v1, v2):
    t = v1.shape[0]
    grid = pl.cdiv(t, ROWS)
    return pl.pallas_call(
        _basis_body,
        grid=(grid,),
        in_specs=[
            pl.BlockSpec((ROWS, 3), lambda i: (i, 0)),
            pl.BlockSpec((ROWS, 3), lambda i: (i, 0)),
        ],
        out_specs=pl.BlockSpec((ROWS, RBF), lambda i: (i, 0)),
        out_shape=jax.ShapeDtypeStruct((t, RBF), jnp.float32),
    )(v1, v2)


def _bond_body(agg_ref, bond_ref, aS_ref, aR_ref,
               wl_ref, bl_ref, wg_ref, bg_ref,
               wb1_ref, wb2_ref, wb3_ref, bb_ref,
               wa_ref, ba_ref, wag_ref, bag_ref,
               bond_out, m_out):
    agg = agg_ref[...]
    bond = bond_ref[...]
    g1 = _silu(_dot(agg, wl_ref[...]) + bl_ref[...])
    g2 = _sig(_dot(agg, wg_ref[...]) + bg_ref[...])
    bond1 = bond + g1 * g2
    pre = (_dot(aS_ref[...], wb1_ref[...]) + _dot(aR_ref[...], wb2_ref[...])
           + _dot(bond1, wb3_ref[...]) + bb_ref[...])
    bond2 = bond1 + _silu(pre)
    m = _silu(_dot(bond2, wa_ref[...]) + ba_ref[...]) * _sig(
        _dot(bond2, wag_ref[...]) + bag_ref[...])
    bond_out[...] = bond2
    m_out[...] = m


def _bond_block(agg, bond, aS, aR, blk):
    e = bond.shape[0]
    grid = pl.cdiv(e, ROWS)
    wb = blk['Wb']
    row = pl.BlockSpec((ROWS, None), lambda i: (i, 0))
    full = lambda r, c: pl.BlockSpec((r, c), lambda i: (0, 0))
    return pl.pallas_call(
        _bond_body,
        grid=(grid,),
        in_specs=[
            pl.BlockSpec((ROWS, RBF), lambda i: (i, 0)),
            pl.BlockSpec((ROWS, UNITS), lambda i: (i, 0)),
            pl.BlockSpec((ROWS, 128), lambda i: (i, 0)),
            pl.BlockSpec((ROWS, 128), lambda i: (i, 0)),
            full(RBF, UNITS), full(1, UNITS),
            full(RBF, UNITS), full(1, UNITS),
            full(128, UNITS), full(128, UNITS), full(UNITS, UNITS),
            full(1, UNITS),
            full(UNITS, UNITS), full(1, UNITS),
            full(UNITS, UNITS), full(1, UNITS),
        ],
        out_specs=[
            pl.BlockSpec((ROWS, UNITS), lambda i: (i, 0)),
            pl.BlockSpec((ROWS, UNITS), lambda i: (i, 0)),
        ],
        out_shape=[
            jax.ShapeDtypeStruct((e, UNITS), jnp.float32),
            jax.ShapeDtypeStruct((e, UNITS), jnp.float32),
        ],
    )(agg, bond, aS, aR,
      blk['Wl'], blk['bl'][None, :], blk['Wg'], blk['bg'][None, :],
      jnp.pad(wb[:UNITS], ((0, 64), (0, 0))),
      jnp.pad(wb[UNITS:2 * UNITS], ((0, 64), (0, 0))),
      wb[2 * UNITS:], blk['bb'][None, :],
      blk['Wa'], blk['ba'][None, :], blk['Wag'], blk['bag'][None, :])


def kernel(atom_positions, atom_types, bond_atom_indices,
           triple_bond_indices, batch_ids, params):
    pos = atom_positions
    n = pos.shape[0]
    e = bond_atom_indices.shape[0]
    t = triple_bond_indices.shape[0]
    perm_t = jnp.argsort(triple_bond_indices[:, 0])
    b1s = triple_bond_indices[perm_t, 0]
    b2s = triple_bond_indices[perm_t, 1]
    sender = bond_atom_indices[:, 0]
    receiver = bond_atom_indices[:, 1]
    vec = pos[receiver] - pos[sender]
    blen = jnp.sqrt(jnp.sum(vec * vec, axis=1, keepdims=True))
    v1 = vec[b1s]
    v2 = vec[b2s]
    tb = _basis(v1, v2)
    apex = receiver[b2s]
    atom = params['emb'][atom_types]
    bond = _silu(blen @ params['Wbp'] + params['bbp'])
    for blk in params['blocks']:
        upd128 = jnp.pad(_sig(atom @ blk['Wu'] + blk['bu']),
                         ((0, 0), (0, 128 - RBF)))
        msg = tb * _sc_gather(upd128, apex)[:, :RBF]
        agg = jax.ops.segment_sum(msg, b1s, num_segments=e,
                                  indices_are_sorted=True)
        atom128 = jnp.pad(atom, ((0, 0), (0, 64)))
        bond, m = _bond_block(agg, bond, _sc_gather(atom128, sender),
                              _sc_gather(atom128, receiver), blk)
        atom = atom + jax.ops.segment_sum(m, receiver, num_segments=n)
    per_atom = atom @ params['Wr'] + params['br']
    energy = jax.ops.segment_sum(per_atom, batch_ids, num_segments=128,
                                 indices_are_sorted=True)
    return energy


# speedup vs baseline: 1.5205x; 1.0437x over previous
import functools

import jax
import jax.numpy as jnp
from jax.experimental import pallas as pl
from jax.experimental.pallas import tpu as pltpu
from jax.experimental.pallas import tpu_sc as plsc

UNITS = 64
MAX_N = 3
SHF = 16
RBF = MAX_N * SHF
CUTOFF = 5.0

ROWS = 4096


@functools.lru_cache(maxsize=None)
def _make_sc_gather(D, B, CH):
    info = plsc.get_sparse_core_info()
    nc, ns = info.num_cores, info.num_subcores
    nw = nc * ns
    assert B % nw == 0
    bpw = B // nw
    assert bpw % CH == 0 and CH % 8 == 0
    nch = bpw // CH
    mesh = plsc.VectorSubcoreMesh(core_axis_name="c", subcore_axis_name="s")

    def gk(table, idx, out, idx_v, rows_v, sem):
        wid = jax.lax.axis_index("s") * nc + jax.lax.axis_index("c")
        base = wid * bpw

        def body(c, carry):
            off = base + c * CH
            pltpu.sync_copy(idx.at[pl.ds(off, CH)], idx_v)
            pltpu.async_copy(table.at[idx_v], rows_v, sem).wait()
            pltpu.sync_copy(rows_v, out.at[pl.ds(off, CH)])
            return carry

        jax.lax.fori_loop(0, nch, body, 0)

    def call(table, idx):
        k = functools.partial(
            pl.kernel, mesh=mesh,
            out_type=jax.ShapeDtypeStruct((B, D), jnp.float32),
            scratch_types=[
                pltpu.VMEM((CH,), jnp.int32),
                pltpu.VMEM((CH, D), jnp.float32),
                pltpu.SemaphoreType.DMA,
            ],
        )(gk)
        return k(table, idx)

    return call


def _sc_gather(table, idx, CH=1000):
    assert table.shape[1] == 128
    return _make_sc_gather(128, idx.shape[0], CH)(table, idx)


def _sig(x):
    return jax.nn.sigmoid(x)


def _silu(x):
    return x * _sig(x)


def _dot(a, b):
    return jax.lax.dot_general(a, b, (((1,), (0,)), ((), ())),
                               preferred_element_type=jnp.float32)


def _basis_body(v1_ref, v2_ref, tb_ref):
    v1 = v1_ref[...]
    v2 = v2_ref[...]
    rows = v1.shape[0]
    l1 = jnp.sqrt(jnp.sum(v1 * v1, axis=1, keepdims=True))
    r = jnp.sqrt(jnp.sum(v2 * v2, axis=1, keepdims=True))
    d12 = jnp.sum(v1 * v2, axis=1, keepdims=True)
    x = jnp.clip(d12 / ((l1 + 1e-8) * (r + 1e-8)), -1.0, 1.0)
    coef = jnp.sqrt(2.0 / CUTOFF)
    s = jnp.pi / CUTOFF
    inv = coef / (r + 1e-8)
    s1 = jnp.sin(s * r)
    c1 = jnp.cos(s * r)
    rad1 = s1 * inv
    rad2 = (2.0 * s1 * c1) * inv
    rad3 = (s1 * (3.0 - 4.0 * s1 * s1)) * inv
    cheb = [jnp.ones_like(x), x]
    for _ in range(SHF - 2):
        cheb.append(2.0 * x * cheb[-1] - cheb[-2])
    k = jax.lax.broadcasted_iota(jnp.int32, (rows, RBF), 1)
    lk = jnp.bitwise_and(k, SHF - 1)
    ang = jnp.broadcast_to(cheb[SHF - 1], (rows, RBF))
    for l in range(SHF - 2, -1, -1):
        ang = jnp.where(lk == l, jnp.broadcast_to(cheb[l], (rows, RBF)), ang)
    rad = jnp.where(k < SHF, jnp.broadcast_to(rad1, (rows, RBF)),
                    jnp.where(k < 2 * SHF,
                              jnp.broadcast_to(rad2, (rows, RBF)),
                              jnp.broadcast_to(rad3, (rows, RBF))))
    tb_ref[...] = rad * ang


def _basis(v1, v2):
    t = v1.shape[0]
    grid = pl.cdiv(t, ROWS)
    return pl.pallas_call(
        _basis_body,
        grid=(grid,),
        in_specs=[
            pl.BlockSpec((ROWS, 3), lambda i: (i, 0)),
            pl.BlockSpec((ROWS, 3), lambda i: (i, 0)),
        ],
        out_specs=pl.BlockSpec((ROWS, RBF), lambda i: (i, 0)),
        out_shape=jax.ShapeDtypeStruct((t, RBF), jnp.float32),
    )(v1, v2)


def _bond_body(agg_ref, bond_ref, aS_ref, aR_ref,
               wl_ref, bl_ref, wg_ref, bg_ref,
               wb1_ref, wb2_ref, wb3_ref, bb_ref,
               wa_ref, ba_ref, wag_ref, bag_ref,
               bond_out, m_out):
    agg = agg_ref[...]
    bond = bond_ref[...]
    g1 = _silu(_dot(agg, wl_ref[...]) + bl_ref[...])
    g2 = _sig(_dot(agg, wg_ref[...]) + bg_ref[...])
    bond1 = bond + g1 * g2
    pre = (_dot(aS_ref[...], wb1_ref[...]) + _dot(aR_ref[...], wb2_ref[...])
           + _dot(bond1, wb3_ref[...]) + bb_ref[...])
    bond2 = bond1 + _silu(pre)
    m = _silu(_dot(bond2, wa_ref[...]) + ba_ref[...]) * _sig(
        _dot(bond2, wag_ref[...]) + bag_ref[...])
    bond_out[...] = bond2
    m_out[...] = m


def _bond_block(agg, bond, aS, aR, blk):
    e = bond.shape[0]
    grid = pl.cdiv(e, ROWS)
    wb = blk['Wb']
    row = pl.BlockSpec((ROWS, None), lambda i: (i, 0))
    full = lambda r, c: pl.BlockSpec((r, c), lambda i: (0, 0))
    return pl.pallas_call(
        _bond_body,
        grid=(grid,),
        in_specs=[
            pl.BlockSpec((ROWS, RBF), lambda i: (i, 0)),
            pl.BlockSpec((ROWS, UNITS), lambda i: (i, 0)),
            pl.BlockSpec((ROWS, 128), lambda i: (i, 0)),
            pl.BlockSpec((ROWS, 128), lambda i: (i, 0)),
            full(RBF, UNITS), full(1, UNITS),
            full(RBF, UNITS), full(1, UNITS),
            full(128, UNITS), full(128, UNITS), full(UNITS, UNITS),
            full(1, UNITS),
            full(UNITS, UNITS), full(1, UNITS),
            full(UNITS, UNITS), full(1, UNITS),
        ],
        out_specs=[
            pl.BlockSpec((ROWS, UNITS), lambda i: (i, 0)),
            pl.BlockSpec((ROWS, UNITS), lambda i: (i, 0)),
        ],
        out_shape=[
            jax.ShapeDtypeStruct((e, UNITS), jnp.float32),
            jax.ShapeDtypeStruct((e, UNITS), jnp.float32),
        ],
    )(agg, bond, aS, aR,
      blk['Wl'], blk['bl'][None, :], blk['Wg'], blk['bg'][None, :],
      jnp.pad(wb[:UNITS], ((0, 64), (0, 0))),
      jnp.pad(wb[UNITS:2 * UNITS], ((0, 64), (0, 0))),
      wb[2 * UNITS:], blk['bb'][None, :],
      blk['Wa'], blk['ba'][None, :], blk['Wag'], blk['bag'][None, :])


def kernel(atom_positions, atom_types, bond_atom_indices,
           triple_bond_indices, batch_ids, params):
    pos = atom_positions
    n = pos.shape[0]
    e = bond_atom_indices.shape[0]
    t = triple_bond_indices.shape[0]
    perm_t = jnp.argsort(triple_bond_indices[:, 0])
    b1s = triple_bond_indices[perm_t, 0]
    b2s = triple_bond_indices[perm_t, 1]
    sender = bond_atom_indices[:, 0]
    receiver = bond_atom_indices[:, 1]
    vec = pos[receiver] - pos[sender]
    blen = jnp.sqrt(jnp.sum(vec * vec, axis=1, keepdims=True))
    v1 = vec[b1s]
    v2 = vec[b2s]
    tb = _basis(v1, v2)
    apex = receiver[b2s]
    atom = params['emb'][atom_types]
    bond = _silu(blen @ params['Wbp'] + params['bbp'])
    for blk in params['blocks']:
        upd128 = jnp.pad(_sig(atom @ blk['Wu'] + blk['bu']),
                         ((0, 0), (0, 128 - RBF)))
        msg = tb * _sc_gather(upd128, apex)[:, :RBF]
        agg = jax.ops.segment_sum(msg, b1s, num_segments=e,
                                  indices_are_sorted=True)
        atom128 = jnp.pad(atom, ((0, 0), (0, 64)))
        bond, m = _bond_block(agg, bond, _sc_gather(atom128, sender),
                              _sc_gather(atom128, receiver), blk)
        atom = atom + jax.ops.segment_sum(m, receiver, num_segments=n)
    per_atom = atom @ params['Wr'] + params['br']
    energy = jax.ops.segment_sum(per_atom, batch_ids, num_segments=128,
                                 indices_are_sorted=True)
    return energy


# speedup vs baseline: 1.6885x; 1.1105x over previous
import functools

import jax
import jax.numpy as jnp
from jax.experimental import pallas as pl
from jax.experimental.pallas import tpu as pltpu
from jax.experimental.pallas import tpu_sc as plsc

UNITS = 64
MAX_N = 3
SHF = 16
RBF = MAX_N * SHF
CUTOFF = 5.0

ROWS = 4096


@functools.lru_cache(maxsize=None)
def _make_sc_gather(D, B, CH):
    info = plsc.get_sparse_core_info()
    nc, ns = info.num_cores, info.num_subcores
    nw = nc * ns
    assert B % nw == 0
    bpw = B // nw
    assert bpw % CH == 0 and CH % 8 == 0
    nch = bpw // CH
    mesh = plsc.VectorSubcoreMesh(core_axis_name="c", subcore_axis_name="s")

    def gk(table, idx, out, idx_v, rows_v, sem):
        wid = jax.lax.axis_index("s") * nc + jax.lax.axis_index("c")
        base = wid * bpw

        def body(c, carry):
            off = base + c * CH
            pltpu.sync_copy(idx.at[pl.ds(off, CH)], idx_v)
            pltpu.async_copy(table.at[idx_v], rows_v, sem).wait()
            pltpu.sync_copy(rows_v, out.at[pl.ds(off, CH)])
            return carry

        jax.lax.fori_loop(0, nch, body, 0)

    def call(table, idx):
        k = functools.partial(
            pl.kernel, mesh=mesh,
            out_type=jax.ShapeDtypeStruct((B, D), jnp.float32),
            scratch_types=[
                pltpu.VMEM((CH,), jnp.int32),
                pltpu.VMEM((CH, D), jnp.float32),
                pltpu.SemaphoreType.DMA,
            ],
        )(gk)
        return k(table, idx)

    return call


def _sc_gather(table, idx, CH=1000):
    assert table.shape[1] == 128
    return _make_sc_gather(128, idx.shape[0], CH)(table, idx)


def _sig(x):
    return jax.nn.sigmoid(x)


def _silu(x):
    return x * _sig(x)


def _dot(a, b):
    return jax.lax.dot_general(a, b, (((1,), (0,)), ((), ())),
                               preferred_element_type=jnp.float32)


def _basis_body(v1_ref, v2_ref, tb_ref):
    v1 = v1_ref[...][:, :3]
    v2 = v2_ref[...][:, :3]
    rows = v1.shape[0]
    l1 = jnp.sqrt(jnp.sum(v1 * v1, axis=1, keepdims=True))
    r = jnp.sqrt(jnp.sum(v2 * v2, axis=1, keepdims=True))
    d12 = jnp.sum(v1 * v2, axis=1, keepdims=True)
    x = jnp.clip(d12 / ((l1 + 1e-8) * (r + 1e-8)), -1.0, 1.0)
    coef = jnp.sqrt(2.0 / CUTOFF)
    s = jnp.pi / CUTOFF
    inv = coef / (r + 1e-8)
    s1 = jnp.sin(s * r)
    c1 = jnp.cos(s * r)
    rad1 = s1 * inv
    rad2 = (2.0 * s1 * c1) * inv
    rad3 = (s1 * (3.0 - 4.0 * s1 * s1)) * inv
    cheb = [jnp.ones_like(x), x]
    for _ in range(SHF - 2):
        cheb.append(2.0 * x * cheb[-1] - cheb[-2])
    k = jax.lax.broadcasted_iota(jnp.int32, (rows, RBF), 1)
    lk = jnp.bitwise_and(k, SHF - 1)
    ang = jnp.broadcast_to(cheb[SHF - 1], (rows, RBF))
    for l in range(SHF - 2, -1, -1):
        ang = jnp.where(lk == l, jnp.broadcast_to(cheb[l], (rows, RBF)), ang)
    rad = jnp.where(k < SHF, jnp.broadcast_to(rad1, (rows, RBF)),
                    jnp.where(k < 2 * SHF,
                              jnp.broadcast_to(rad2, (rows, RBF)),
                              jnp.broadcast_to(rad3, (rows, RBF))))
    tb_ref[...] = rad * ang


def _basis(v1, v2):
    t = v1.shape[0]
    grid = pl.cdiv(t, ROWS)
    return pl.pallas_call(
        _basis_body,
        grid=(grid,),
        in_specs=[
            pl.BlockSpec((ROWS, 128), lambda i: (i, 0)),
            pl.BlockSpec((ROWS, 128), lambda i: (i, 0)),
        ],
        out_specs=pl.BlockSpec((ROWS, RBF), lambda i: (i, 0)),
        out_shape=jax.ShapeDtypeStruct((t, RBF), jnp.float32),
    )(v1, v2)


def _bond_body(agg_ref, bond_ref, aS_ref, aR_ref,
               wl_ref, bl_ref, wg_ref, bg_ref,
               wb1_ref, wb2_ref, wb3_ref, bb_ref,
               wa_ref, ba_ref, wag_ref, bag_ref,
               bond_out, m_out):
    agg = agg_ref[...]
    bond = bond_ref[...]
    g1 = _silu(_dot(agg, wl_ref[...]) + bl_ref[...])
    g2 = _sig(_dot(agg, wg_ref[...]) + bg_ref[...])
    bond1 = bond + g1 * g2
    pre = (_dot(aS_ref[...], wb1_ref[...]) + _dot(aR_ref[...], wb2_ref[...])
           + _dot(bond1, wb3_ref[...]) + bb_ref[...])
    bond2 = bond1 + _silu(pre)
    m = _silu(_dot(bond2, wa_ref[...]) + ba_ref[...]) * _sig(
        _dot(bond2, wag_ref[...]) + bag_ref[...])
    bond_out[...] = bond2
    m_out[...] = m


def _bond_block(agg, bond, aS, aR, blk):
    e = bond.shape[0]
    grid = pl.cdiv(e, ROWS)
    wb = blk['Wb']
    row = pl.BlockSpec((ROWS, None), lambda i: (i, 0))
    full = lambda r, c: pl.BlockSpec((r, c), lambda i: (0, 0))
    return pl.pallas_call(
        _bond_body,
        grid=(grid,),
        in_specs=[
            pl.BlockSpec((ROWS, RBF), lambda i: (i, 0)),
            pl.BlockSpec((ROWS, UNITS), lambda i: (i, 0)),
            pl.BlockSpec((ROWS, 128), lambda i: (i, 0)),
            pl.BlockSpec((ROWS, 128), lambda i: (i, 0)),
            full(RBF, UNITS), full(1, UNITS),
            full(RBF, UNITS), full(1, UNITS),
            full(128, UNITS), full(128, UNITS), full(UNITS, UNITS),
            full(1, UNITS),
            full(UNITS, UNITS), full(1, UNITS),
            full(UNITS, UNITS), full(1, UNITS),
        ],
        out_specs=[
            pl.BlockSpec((ROWS, UNITS), lambda i: (i, 0)),
            pl.BlockSpec((ROWS, UNITS), lambda i: (i, 0)),
        ],
        out_shape=[
            jax.ShapeDtypeStruct((e, UNITS), jnp.float32),
            jax.ShapeDtypeStruct((e, UNITS), jnp.float32),
        ],
    )(agg, bond, aS, aR,
      blk['Wl'], blk['bl'][None, :], blk['Wg'], blk['bg'][None, :],
      jnp.pad(wb[:UNITS], ((0, 64), (0, 0))),
      jnp.pad(wb[UNITS:2 * UNITS], ((0, 64), (0, 0))),
      wb[2 * UNITS:], blk['bb'][None, :],
      blk['Wa'], blk['ba'][None, :], blk['Wag'], blk['bag'][None, :])


def kernel(atom_positions, atom_types, bond_atom_indices,
           triple_bond_indices, batch_ids, params):
    pos = atom_positions
    n = pos.shape[0]
    e = bond_atom_indices.shape[0]
    t = triple_bond_indices.shape[0]
    perm_t = jnp.argsort(triple_bond_indices[:, 0])
    b1s = triple_bond_indices[perm_t, 0]
    b2s = triple_bond_indices[perm_t, 1]
    sender = bond_atom_indices[:, 0]
    receiver = bond_atom_indices[:, 1]
    pos128 = jnp.pad(pos, ((0, 0), (0, 125)))
    vec128 = _sc_gather(pos128, receiver) - _sc_gather(pos128, sender)
    blen = jnp.sqrt(jnp.sum(vec128[:, :3] * vec128[:, :3], axis=1,
                            keepdims=True))
    v1 = _sc_gather(vec128, b1s)
    v2 = _sc_gather(vec128, b2s)
    tb = _basis(v1, v2)
    apex = receiver[b2s]
    atom = params['emb'][atom_types]
    bond = _silu(blen @ params['Wbp'] + params['bbp'])
    for blk in params['blocks']:
        upd128 = jnp.pad(_sig(atom @ blk['Wu'] + blk['bu']),
                         ((0, 0), (0, 128 - RBF)))
        msg = tb * _sc_gather(upd128, apex)[:, :RBF]
        agg = jax.ops.segment_sum(msg, b1s, num_segments=e,
                                  indices_are_sorted=True)
        atom128 = jnp.pad(atom, ((0, 0), (0, 64)))
        bond, m = _bond_block(agg, bond, _sc_gather(atom128, sender),
                              _sc_gather(atom128, receiver), blk)
        atom = atom + jax.ops.segment_sum(m, receiver, num_segments=n)
    per_atom = atom @ params['Wr'] + params['br']
    energy = jax.ops.segment_sum(per_atom, batch_ids, num_segments=128,
                                 indices_are_sorted=True)
    return energy
